# Initial kernel scaffold; baseline (speedup 1.0000x reference)
import jax
import jax.numpy as jnp
from jax.experimental import pallas as pl


def kernel(feat, edge_index, edge_weight):
    raise NotImplementedError("write your pallas kernel here")


# speedup vs baseline: 3.2372x; 3.2372x over previous
import jax
import jax.numpy as jnp
from jax import lax
from jax.experimental import pallas as pl
from jax.experimental.pallas import tpu as pltpu
from jax.experimental.pallas import tpu_sc as plsc

N = 10000
E = 320000
D = 128
DH = D // 2
NT = 16
C = 128
NCHUNK = 160
EPT = NCHUNK * C
EP = NT * EPT
NP = 10240
RPT = NP // NT


def _sc_body(feat2, gidx3, sidx3, w3, out, acc, gidx, sidx, wv, rows, sem):
    c = lax.axis_index("c")
    s = lax.axis_index("s")

    pltpu.sync_copy(gidx3.at[c, s], gidx)
    pltpu.sync_copy(sidx3.at[s], sidx)
    pltpu.sync_copy(w3.at[s], wv)

    zero16 = jnp.zeros((16,), jnp.float32)

    def _zrow(i, carry):
        for j in range(DH // 16):
            rows[i, pl.ds(j * 16, 16)] = zero16
        return carry

    lax.fori_loop(0, C, _zrow, 0)
    for t in range(RPT // C):
        pltpu.sync_copy(rows, acc.at[pl.ds(s * RPT + t * C, C)])
    plsc.subcore_barrier()

    def _chunk(k, carry):
        pltpu.async_copy(feat2.at[gidx.at[k]], rows, sem).wait()

        def _sgroup(g, cc):
            w16 = wv[k, pl.ds(g * 16, 16)]
            for j in range(16):
                w = w16[j]
                r = g * 16 + j
                for q in range(DH // 16):
                    rows[r, pl.ds(q * 16, 16)] = rows[r, pl.ds(q * 16, 16)] * w
            return cc

        lax.fori_loop(0, C // 16, _sgroup, 0)
        pltpu.sync_copy(rows, acc.at[sidx.at[k]], add=True)
        return carry

    lax.fori_loop(0, NCHUNK, _chunk, 0)
    plsc.subcore_barrier()

    for t in range(RPT // C):
        pltpu.sync_copy(acc.at[pl.ds(s * RPT + t * C, C)], rows)
        pltpu.sync_copy(rows, out.at[c, pl.ds(s * RPT + t * C, C)])


_sc_call = pl.kernel(
    _sc_body,
    out_type=jax.ShapeDtypeStruct((2, NP, DH), jnp.float32),
    mesh=plsc.VectorSubcoreMesh(core_axis_name="c", subcore_axis_name="s"),
    compiler_params=pltpu.CompilerParams(use_tc_tiling_on_sc=False),
    scratch_types=[
        pltpu.VMEM_SHARED((NP, DH), jnp.float32),
        pltpu.VMEM((NCHUNK, C), jnp.int32),
        pltpu.VMEM((NCHUNK, C), jnp.int32),
        pltpu.VMEM((NCHUNK, C), jnp.float32),
        pltpu.VMEM((C, DH), jnp.float32),
        pltpu.SemaphoreType.DMA,
    ],
)


@jax.jit
def kernel(feat, edge_index, edge_weight):
    feat2 = jnp.concatenate([feat[:, :DH], feat[:, DH:]], axis=0)
    src = edge_index[0].astype(jnp.int32)
    dst = edge_index[1].astype(jnp.int32)
    pad = EP - E
    pad_idx = jnp.arange(pad, dtype=jnp.int32) % N
    src_p = jnp.concatenate([src, pad_idx])
    dst_p = jnp.concatenate([dst, pad_idx])
    w_p = jnp.concatenate(
        [edge_weight, jnp.zeros((pad,), jnp.float32)])
    gidx3 = jnp.stack([dst_p, dst_p + N]).reshape(2, NT, NCHUNK, C)
    sidx3 = src_p.reshape(NT, NCHUNK, C)
    w3 = w_p.reshape(NT, NCHUNK, C)
    out2 = _sc_call(feat2, gidx3, sidx3, w3)
    return jnp.concatenate([out2[0, :N], out2[1, :N]], axis=1)


# speedup vs baseline: 4.1726x; 1.2890x over previous
import jax
import jax.numpy as jnp
from jax import lax
from jax.experimental import pallas as pl
from jax.experimental.pallas import tpu as pltpu
from jax.experimental.pallas import tpu_sc as plsc

N = 10000
E = 320000
D = 128
DH = D // 2
NT = 16
C = 128
NCHUNK = 159
EPT = NCHUNK * C
EP = NT * EPT
NP = 10240
RPT = NP // NT
NBUF = 3
AHEAD = 2


def _sc_body(feat2, gidx3, sidx3, w3, out, acc, gidx, sidx, wv,
             rows0, rows1, rows2, sem0, sem1, sem2):
    c = lax.axis_index("c")
    s = lax.axis_index("s")
    bufs = (rows0, rows1, rows2)
    sems = (sem0, sem1, sem2)

    pltpu.sync_copy(gidx3.at[c, s], gidx)
    pltpu.sync_copy(sidx3.at[s], sidx)
    pltpu.sync_copy(w3.at[s], wv)

    zero16 = jnp.zeros((16,), jnp.float32)

    def _zrow(i, carry):
        for j in range(DH // 16):
            rows0[i, pl.ds(j * 16, 16)] = zero16
        return carry

    lax.fori_loop(0, C, _zrow, 0)
    for t in range(RPT // C):
        pltpu.sync_copy(rows0, acc.at[pl.ds(s * RPT + t * C, C)])
    plsc.subcore_barrier()

    def _scale(k, rows):
        def _sgroup(g, cc):
            w16 = wv[k, pl.ds(g * 16, 16)]
            for j in range(16):
                w = w16[j]
                r = g * 16 + j
                for q in range(DH // 16):
                    rows[r, pl.ds(q * 16, 16)] = rows[r, pl.ds(q * 16, 16)] * w
            return cc

        lax.fori_loop(0, C // 16, _sgroup, 0)

    for k in range(AHEAD):
        pltpu.async_copy(feat2.at[gidx.at[k]], bufs[k], sems[k])

    def _quad(qi, carry):
        k0 = NBUF * qi
        for b in range(NBUF):
            k = k0 + b
            nb = (b + AHEAD) % NBUF

            @pl.when(k + AHEAD < NCHUNK)
            def _prefetch():
                pltpu.async_copy(
                    feat2.at[gidx.at[k + AHEAD]], bufs[nb], sems[nb])

            pltpu.make_async_copy(
                feat2.at[gidx.at[k]], bufs[b], sems[b]).wait()
            _scale(k, bufs[b])
            pltpu.sync_copy(bufs[b], acc.at[sidx.at[k]], add=True)
        return carry

    lax.fori_loop(0, NCHUNK // NBUF, _quad, 0)
    plsc.subcore_barrier()

    for t in range(RPT // C):
        pltpu.sync_copy(acc.at[pl.ds(s * RPT + t * C, C)], rows0)
        pltpu.sync_copy(rows0, out.at[c, pl.ds(s * RPT + t * C, C)])


_sc_call = pl.kernel(
    _sc_body,
    out_type=jax.ShapeDtypeStruct((2, NP, DH), jnp.float32),
    mesh=plsc.VectorSubcoreMesh(core_axis_name="c", subcore_axis_name="s"),
    compiler_params=pltpu.CompilerParams(use_tc_tiling_on_sc=False),
    scratch_types=[
        pltpu.VMEM_SHARED((NP, DH), jnp.float32),
        pltpu.VMEM((NCHUNK, C), jnp.int32),
        pltpu.VMEM((NCHUNK, C), jnp.int32),
        pltpu.VMEM((NCHUNK, C), jnp.float32),
        pltpu.VMEM((C, DH), jnp.float32),
        pltpu.VMEM((C, DH), jnp.float32),
        pltpu.VMEM((C, DH), jnp.float32),
        pltpu.SemaphoreType.DMA,
        pltpu.SemaphoreType.DMA,
        pltpu.SemaphoreType.DMA,
    ],
)


@jax.jit
def kernel(feat, edge_index, edge_weight):
    feat2 = jnp.concatenate([feat[:, :DH], feat[:, DH:]], axis=0)
    src = edge_index[0].astype(jnp.int32)
    dst = edge_index[1].astype(jnp.int32)
    pad = EP - E
    pad_idx = jnp.arange(pad, dtype=jnp.int32) % N
    src_p = jnp.concatenate([src, pad_idx])
    dst_p = jnp.concatenate([dst, pad_idx])
    w_p = jnp.concatenate(
        [edge_weight, jnp.zeros((pad,), jnp.float32)])
    gidx3 = jnp.stack([dst_p, dst_p + N]).reshape(2, NT, NCHUNK, C)
    sidx3 = src_p.reshape(NT, NCHUNK, C)
    w3 = w_p.reshape(NT, NCHUNK, C)
    out2 = _sc_call(feat2, gidx3, sidx3, w3)
    return jnp.concatenate([out2[0, :N], out2[1, :N]], axis=1)


# speedup vs baseline: 9.2633x; 2.2200x over previous
import jax
import jax.numpy as jnp
from jax import lax
from jax.experimental import pallas as pl
from jax.experimental.pallas import tpu as pltpu
from jax.experimental.pallas import tpu_sc as plsc

N = 10000
E = 320000
D = 128
DH = D // 2
NT = 16
C = 128
NCHUNK = 159
EPT = NCHUNK * C
EP = NT * EPT
NP = 10240
RPT = NP // NT
NBUF = 3
AHEAD = 2


def _sc_body(feat2, gidx3, sidx3, w3, out, acc, gidx, sidx, wv,
             rows0, rows1, rows2, sem0, sem1, sem2, ssem0, ssem1, ssem2):
    c = lax.axis_index("c")
    s = lax.axis_index("s")
    bufs = (rows0, rows1, rows2)
    sems = (sem0, sem1, sem2)
    ssems = (ssem0, ssem1, ssem2)

    pltpu.sync_copy(gidx3.at[c, s], gidx)
    pltpu.sync_copy(sidx3.at[s], sidx)
    pltpu.sync_copy(w3.at[s], wv)

    zero16 = jnp.zeros((16,), jnp.float32)

    def _zrow(i, carry):
        for j in range(DH // 16):
            rows0[i, pl.ds(j * 16, 16)] = zero16
        return carry

    lax.fori_loop(0, C, _zrow, 0)
    for t in range(RPT // C):
        pltpu.sync_copy(rows0, acc.at[pl.ds(s * RPT + t * C, C)])
    plsc.subcore_barrier()

    def _scale(k, rows):
        @plsc.parallel_loop(0, C, unroll=8,
                            carry=jnp.full((16,), k * C, jnp.int32))
        def _row(r, widx):
            w = plsc.load_gather(wv, [widx])
            for q in range(DH // 16):
                rows[r, pl.ds(q * 16, 16)] = rows[r, pl.ds(q * 16, 16)] * w
            return widx + 1

    for k in range(AHEAD):
        pltpu.async_copy(feat2.at[gidx.at[k]], bufs[k], sems[k])

    def _quad(qi, carry):
        k0 = NBUF * qi
        for b in range(NBUF):
            k = k0 + b
            nb = (b + AHEAD) % NBUF

            @pl.when((k + AHEAD < NCHUNK) & (k >= 1))
            def _drain_scatter():
                pltpu.make_async_copy(
                    bufs[nb], acc.at[sidx.at[k - 1]], ssems[nb]).wait()

            @pl.when(k + AHEAD < NCHUNK)
            def _prefetch():
                pltpu.async_copy(
                    feat2.at[gidx.at[k + AHEAD]], bufs[nb], sems[nb])

            pltpu.make_async_copy(
                feat2.at[gidx.at[k]], bufs[b], sems[b]).wait()
            _scale(k, bufs[b])
            pltpu.async_copy(bufs[b], acc.at[sidx.at[k]], ssems[b], add=True)
        return carry

    lax.fori_loop(0, NCHUNK // NBUF, _quad, 0)
    for b in range(NBUF):
        kf = NCHUNK - NBUF + b
        pltpu.make_async_copy(
            bufs[kf % NBUF], acc.at[sidx.at[kf]], ssems[kf % NBUF]).wait()
    plsc.subcore_barrier()

    for t in range(RPT // C):
        pltpu.sync_copy(acc.at[pl.ds(s * RPT + t * C, C)], rows0)
        pltpu.sync_copy(rows0, out.at[c, pl.ds(s * RPT + t * C, C)])


_sc_call = pl.kernel(
    _sc_body,
    out_type=jax.ShapeDtypeStruct((2, NP, DH), jnp.float32),
    mesh=plsc.VectorSubcoreMesh(core_axis_name="c", subcore_axis_name="s"),
    compiler_params=pltpu.CompilerParams(
        use_tc_tiling_on_sc=False, needs_layout_passes=False),
    scratch_types=[
        pltpu.VMEM_SHARED((NP, DH), jnp.float32),
        pltpu.VMEM((NCHUNK, C), jnp.int32),
        pltpu.VMEM((NCHUNK, C), jnp.int32),
        pltpu.VMEM((NCHUNK * C,), jnp.float32),
        pltpu.VMEM((C, DH), jnp.float32),
        pltpu.VMEM((C, DH), jnp.float32),
        pltpu.VMEM((C, DH), jnp.float32),
        pltpu.SemaphoreType.DMA,
        pltpu.SemaphoreType.DMA,
        pltpu.SemaphoreType.DMA,
        pltpu.SemaphoreType.DMA,
        pltpu.SemaphoreType.DMA,
        pltpu.SemaphoreType.DMA,
    ],
)


@jax.jit
def kernel(feat, edge_index, edge_weight):
    feat2 = jnp.concatenate([feat[:, :DH], feat[:, DH:]], axis=0)
    src = edge_index[0].astype(jnp.int32)
    dst = edge_index[1].astype(jnp.int32)
    pad = EP - E
    pad_idx = jnp.arange(pad, dtype=jnp.int32) % N
    src_p = jnp.concatenate([src, pad_idx])
    dst_p = jnp.concatenate([dst, pad_idx])
    w_p = jnp.concatenate(
        [edge_weight, jnp.zeros((pad,), jnp.float32)])
    gidx3 = jnp.stack([dst_p, dst_p + N]).reshape(2, NT, NCHUNK, C)
    sidx3 = src_p.reshape(NT, NCHUNK, C)
    w3 = w_p.reshape(NT, NCHUNK * C)
    out2 = _sc_call(feat2, gidx3, sidx3, w3)
    return jnp.concatenate([out2[0, :N], out2[1, :N]], axis=1)


# speedup vs baseline: 9.6335x; 1.0400x over previous
import jax
import jax.numpy as jnp
from jax import lax
from jax.experimental import pallas as pl
from jax.experimental.pallas import tpu as pltpu
from jax.experimental.pallas import tpu_sc as plsc

N = 10000
E = 320000
D = 128
DH = D // 2
NT = 16
C = 128
NCHUNK = 162
EPT = NCHUNK * C
EP = NT * EPT
NP = 10240
RPT = NP // NT
NRF = N // NT
NBUF = 3
NIB = 6
AHEAD = 2
REC = 3 * C


def _sc_body(feat_h, pack_h, out, acc, feat_s,
             rows0, rows1, rows2, sem0, sem1, sem2, ssem0, ssem1, ssem2,
             ib0, ib1, ib2, ib3, ib4, ib5,
             isem0, isem1, isem2, isem3, isem4, isem5):
    c = lax.axis_index("c")
    s = lax.axis_index("s")
    bufs = (rows0, rows1, rows2)
    sems = (sem0, sem1, sem2)
    ssems = (ssem0, ssem1, ssem2)
    ibs = (ib0, ib1, ib2, ib3, ib4, ib5)
    isems = (isem0, isem1, isem2, isem3, isem4, isem5)

    pltpu.sync_copy(feat_h.at[c, pl.ds(s * NRF, NRF)],
                    feat_s.at[pl.ds(s * NRF, NRF)])

    zero16 = jnp.zeros((16,), jnp.float32)

    def _zrow(i, carry):
        for j in range(DH // 16):
            rows0[i, pl.ds(j * 16, 16)] = zero16
        return carry

    lax.fori_loop(0, C, _zrow, 0)
    for t in range(RPT // C):
        pltpu.sync_copy(rows0, acc.at[pl.ds(s * RPT + t * C, C)])
    plsc.subcore_barrier()

    def _gidx(m):
        return ibs[m].at[pl.ds(0, C)]

    def _sidx(m):
        return ibs[m].at[pl.ds(C, C)]

    def _scale(rows, ib):
        @plsc.parallel_loop(0, C, unroll=8,
                            carry=jnp.full((16,), 2 * C, jnp.int32))
        def _row(r, widx):
            w = plsc.bitcast(plsc.load_gather(ib, [widx]), jnp.float32)
            for q in range(DH // 16):
                rows[r, pl.ds(q * 16, 16)] = rows[r, pl.ds(q * 16, 16)] * w
            return widx + 1

    for j in range(NIB - 1):
        pltpu.async_copy(pack_h.at[s, j], ibs[j], isems[j])
    for k in range(AHEAD):
        pltpu.make_async_copy(pack_h.at[s, k], ibs[k], isems[k]).wait()
        pltpu.async_copy(feat_s.at[_gidx(k)], bufs[k], sems[k])

    def _hex(qi, carry):
        k0 = NIB * qi
        for u in range(NIB):
            k = k0 + u
            b = u % NBUF
            pb = (u - 1) % NBUF
            pm = (u - 1) % NIB
            nb = (u + AHEAD) % NBUF
            nm = (u + AHEAD) % NIB
            fm = (u + NIB - 1) % NIB

            @pl.when(k >= 1)
            def _drain_scatter():
                pltpu.make_async_copy(
                    bufs[pb], acc.at[_sidx(pm)], ssems[pb]).wait()

            @pl.when(k + NIB - 1 < NCHUNK)
            def _fetch_idx():
                pltpu.async_copy(
                    pack_h.at[s, k + NIB - 1], ibs[fm], isems[fm])

            @pl.when(k + AHEAD < NCHUNK)
            def _prefetch():
                pltpu.make_async_copy(
                    pack_h.at[s, k + AHEAD], ibs[nm], isems[nm]).wait()
                pltpu.async_copy(
                    feat_s.at[_gidx(nm)], bufs[nb], sems[nb])

            pltpu.make_async_copy(
                feat_s.at[_gidx(u)], bufs[b], sems[b]).wait()
            _scale(bufs[b], ibs[u])
            pltpu.async_copy(bufs[b], acc.at[_sidx(u)], ssems[b], add=True)
        return carry

    lax.fori_loop(0, NCHUNK // NIB, _hex, 0)
    pltpu.make_async_copy(
        bufs[(NCHUNK - 1) % NBUF], acc.at[_sidx((NCHUNK - 1) % NIB)],
        ssems[(NCHUNK - 1) % NBUF]).wait()
    plsc.subcore_barrier()

    for t in range(RPT // C):
        pltpu.sync_copy(acc.at[pl.ds(s * RPT + t * C, C)], rows0)
        pltpu.sync_copy(rows0, out.at[c, pl.ds(s * RPT + t * C, C)])


_sc_call = pl.kernel(
    _sc_body,
    out_type=jax.ShapeDtypeStruct((2, NP, DH), jnp.float32),
    mesh=plsc.VectorSubcoreMesh(core_axis_name="c", subcore_axis_name="s"),
    compiler_params=pltpu.CompilerParams(
        use_tc_tiling_on_sc=False, needs_layout_passes=False),
    scratch_types=(
        [pltpu.VMEM_SHARED((NP, DH), jnp.float32)]
        + [pltpu.VMEM_SHARED((N, DH), jnp.float32)]
        + [pltpu.VMEM((C, DH), jnp.float32)] * NBUF
        + [pltpu.SemaphoreType.DMA] * NBUF
        + [pltpu.SemaphoreType.DMA] * NBUF
        + [pltpu.VMEM((REC,), jnp.int32)] * NIB
        + [pltpu.SemaphoreType.DMA] * NIB
    ),
)


@jax.jit
def kernel(feat, edge_index, edge_weight):
    feat_h = jnp.stack([feat[:, :DH], feat[:, DH:]], axis=0)
    src = edge_index[0].astype(jnp.int32)
    dst = edge_index[1].astype(jnp.int32)
    pad = EP - E
    pad_idx = jnp.arange(pad, dtype=jnp.int32) % N
    src_p = jnp.concatenate([src, pad_idx])
    dst_p = jnp.concatenate([dst, pad_idx])
    w_bits = lax.bitcast_convert_type(
        jnp.concatenate([edge_weight, jnp.zeros((pad,), jnp.float32)]),
        jnp.int32)
    pack_h = jnp.stack(
        [dst_p.reshape(NT, NCHUNK, C), src_p.reshape(NT, NCHUNK, C),
         w_bits.reshape(NT, NCHUNK, C)], axis=2).reshape(NT, NCHUNK, REC)
    out2 = _sc_call(feat_h, pack_h)
    return jnp.concatenate([out2[0, :N], out2[1, :N]], axis=1)


# speedup vs baseline: 11.3016x; 1.1732x over previous
import jax
import jax.numpy as jnp
from jax import lax
from jax.experimental import pallas as pl
from jax.experimental.pallas import tpu as pltpu
from jax.experimental.pallas import tpu_sc as plsc

N = 10000
E = 320000
D = 128
NT = 16
C = 128
NCHUNK = 80
EPT = NCHUNK * C
EP = 2 * NT * EPT
NP = 10240
RPT = NP // NT
NBUF = 2
NIB = 4
REC = 3 * C


def _sc_body(feat_h, pack_h, out, acc,
             rows0, rows1, sem0, sem1, ssem0, ssem1,
             ib0, ib1, ib2, ib3, isem0, isem1, isem2, isem3):
    c = lax.axis_index("c")
    s = lax.axis_index("s")
    bufs = (rows0, rows1)
    sems = (sem0, sem1)
    ssems = (ssem0, ssem1)
    ibs = (ib0, ib1, ib2, ib3)
    isems = (isem0, isem1, isem2, isem3)

    zero16 = jnp.zeros((16,), jnp.float32)

    def _zrow(i, carry):
        for j in range(D // 16):
            rows0[i, pl.ds(j * 16, 16)] = zero16
        return carry

    lax.fori_loop(0, C, _zrow, 0)
    for t in range(RPT // C):
        pltpu.sync_copy(rows0, acc.at[pl.ds(s * RPT + t * C, C)])
    plsc.subcore_barrier()

    def _gidx(m):
        return ibs[m].at[pl.ds(0, C)]

    def _sidx(m):
        return ibs[m].at[pl.ds(C, C)]

    def _scale(rows, ib):
        @plsc.parallel_loop(0, C, unroll=8,
                            carry=jnp.full((16,), 2 * C, jnp.int32))
        def _row(r, widx):
            w = plsc.bitcast(plsc.load_gather(ib, [widx]), jnp.float32)
            for q in range(D // 16):
                rows[r, pl.ds(q * 16, 16)] = rows[r, pl.ds(q * 16, 16)] * w
            return widx + 1

    for j in range(NIB - 1):
        pltpu.async_copy(pack_h.at[c, s, j], ibs[j], isems[j])
    pltpu.make_async_copy(pack_h.at[c, s, 0], ibs[0], isems[0]).wait()
    pltpu.async_copy(feat_h.at[_gidx(0)], bufs[0], sems[0])

    def _quad(qi, carry):
        k0 = NIB * qi
        for u in range(NIB):
            k = k0 + u
            b = u % NBUF
            pb = (u - 1) % NBUF
            pm = (u - 1) % NIB
            nb = (u + 1) % NBUF
            nm = (u + 1) % NIB
            fm = (u + NIB - 1) % NIB

            @pl.when(k >= 1)
            def _drain_scatter():
                pltpu.make_async_copy(
                    bufs[pb], acc.at[_sidx(pm)], ssems[pb]).wait()

            @pl.when(k + NIB - 1 < NCHUNK)
            def _fetch_idx():
                pltpu.async_copy(
                    pack_h.at[c, s, k + NIB - 1], ibs[fm], isems[fm])

            @pl.when(k + 1 < NCHUNK)
            def _prefetch():
                pltpu.make_async_copy(
                    pack_h.at[c, s, k + 1], ibs[nm], isems[nm]).wait()
                pltpu.async_copy(
                    feat_h.at[_gidx(nm)], bufs[nb], sems[nb])

            pltpu.make_async_copy(
                feat_h.at[_gidx(u)], bufs[b], sems[b]).wait()
            _scale(bufs[b], ibs[u])
            pltpu.async_copy(bufs[b], acc.at[_sidx(u)], ssems[b], add=True)
        return carry

    lax.fori_loop(0, NCHUNK // NIB, _quad, 0)
    pltpu.make_async_copy(
        bufs[(NCHUNK - 1) % NBUF], acc.at[_sidx((NCHUNK - 1) % NIB)],
        ssems[(NCHUNK - 1) % NBUF]).wait()
    plsc.subcore_barrier()

    for t in range(RPT // C):
        pltpu.sync_copy(acc.at[pl.ds(s * RPT + t * C, C)], rows0)
        pltpu.sync_copy(rows0, out.at[c, pl.ds(s * RPT + t * C, C)])


_sc_call = pl.kernel(
    _sc_body,
    out_type=jax.ShapeDtypeStruct((2, NP, D), jnp.float32),
    mesh=plsc.VectorSubcoreMesh(core_axis_name="c", subcore_axis_name="s"),
    compiler_params=pltpu.CompilerParams(
        use_tc_tiling_on_sc=False, needs_layout_passes=False),
    scratch_types=(
        [pltpu.VMEM_SHARED((NP, D), jnp.float32)]
        + [pltpu.VMEM((C, D), jnp.float32)] * NBUF
        + [pltpu.SemaphoreType.DMA] * NBUF
        + [pltpu.SemaphoreType.DMA] * NBUF
        + [pltpu.VMEM((REC,), jnp.int32)] * NIB
        + [pltpu.SemaphoreType.DMA] * NIB
    ),
)


def _combine_body(p_ref, o_ref):
    o_ref[...] = p_ref[0] + p_ref[1]


_combine = pl.pallas_call(
    _combine_body,
    out_shape=jax.ShapeDtypeStruct((NP, D), jnp.float32),
)


@jax.jit
def kernel(feat, edge_index, edge_weight):
    src = edge_index[0].astype(jnp.int32)
    dst = edge_index[1].astype(jnp.int32)
    pad = EP - E
    pad_idx = jnp.arange(pad, dtype=jnp.int32) % N
    src_p = jnp.concatenate([src, pad_idx])
    dst_p = jnp.concatenate([dst, pad_idx])
    w_bits = lax.bitcast_convert_type(
        jnp.concatenate([edge_weight, jnp.zeros((pad,), jnp.float32)]),
        jnp.int32)
    pack_h = jnp.stack(
        [dst_p.reshape(2, NT, NCHUNK, C), src_p.reshape(2, NT, NCHUNK, C),
         w_bits.reshape(2, NT, NCHUNK, C)], axis=3).reshape(2, NT, NCHUNK, REC)
    partials = _sc_call(feat, pack_h)
    return _combine(partials)[:N]
